# Initial kernel scaffold; baseline (speedup 1.0000x reference)
import jax
import jax.numpy as jnp
from jax.experimental import pallas as pl


def kernel(x, edge_index, num_iter, W1, b1, W2, b2):
    raise NotImplementedError("write your pallas kernel here")


# speedup vs baseline: 5.6052x; 5.6052x over previous
import functools

import jax
import jax.numpy as jnp
from jax import lax
from jax.experimental import pallas as pl
from jax.experimental.pallas import tpu as pltpu
from jax.experimental.pallas import tpu_sc as plsc

K = 50
CLUSTER_TEMP = 30.0
N = 10000
NFEAT = 256
NHID = 512
NOUT = 128

NC, NS, L = 2, 16, 16
CHUNK = 128
NACC = 10240
ROWS_PER_TILE = NACC // NS


def _sc_mesh():
    return plsc.VectorSubcoreMesh(
        core_axis_name="c", subcore_axis_name="s", num_cores=NC, num_subcores=NS)


def _sc_degree(dst3):
    nch = dst3.shape[1]

    @functools.partial(
        pl.kernel,
        out_type=jax.ShapeDtypeStruct((NC, NACC), jnp.float32),
        mesh=_sc_mesh(),
        scratch_types=[
            pltpu.VMEM((nch, CHUNK), jnp.int32),
            pltpu.VMEM((CHUNK,), jnp.float32),
            pltpu.VMEM((ROWS_PER_TILE,), jnp.float32),
            pltpu.VMEM_SHARED((NACC,), jnp.float32),
        ],
    )
    def deg_kernel(dst_hbm, out_hbm, dst_v, ones_v, zer_v, hist_sh):
        c = lax.axis_index("c")
        s = lax.axis_index("s")
        tid = c * NS + s
        pltpu.sync_copy(dst_hbm.at[tid], dst_v)
        ones16 = jnp.ones((L,), jnp.float32)
        for i in range(CHUNK // L):
            ones_v[pl.ds(i * L, L)] = ones16
        z16 = jnp.zeros((L,), jnp.float32)
        for i in range(ROWS_PER_TILE // L):
            zer_v[pl.ds(i * L, L)] = z16
        pltpu.sync_copy(zer_v, hist_sh.at[pl.ds(s * ROWS_PER_TILE, ROWS_PER_TILE)])
        plsc.subcore_barrier()

        def body(j, carry):
            pltpu.sync_copy(ones_v, hist_sh.at[dst_v.at[j]], add=True)
            return carry

        lax.fori_loop(0, nch, body, 0)
        plsc.subcore_barrier()
        pltpu.sync_copy(
            hist_sh.at[pl.ds(s * ROWS_PER_TILE, ROWS_PER_TILE)],
            out_hbm.at[c].at[pl.ds(s * ROWS_PER_TILE, ROWS_PER_TILE)])

    return deg_kernel(dst3)


_GRP = 8


def _sc_aggregate(table2, src3, dst3, edge_split=False, nblk=None):
    cb = table2.shape[-1]
    nch = src3.shape[1]
    if nblk is None:
        nblk = NC
    npass = 1 if edge_split else nblk // NC
    ngrp = nch // _GRP
    zrows = ROWS_PER_TILE // CHUNK

    @functools.partial(
        pl.kernel,
        out_type=jax.ShapeDtypeStruct((nblk, NACC, cb), jnp.float32),
        mesh=_sc_mesh(),
        scratch_types=[
            pltpu.VMEM((_GRP, CHUNK), jnp.int32),
            pltpu.VMEM((_GRP, CHUNK), jnp.int32),
            pltpu.VMEM((2, CHUNK, cb), jnp.float32),
            pltpu.VMEM_SHARED((NACC, cb), jnp.float32),
            pltpu.SemaphoreType.DMA((2,)),
        ],
    )
    def agg_kernel(tab_hbm, src_hbm, dst_hbm, out_hbm, src_v, dst_v, rows_v,
                   acc_sh, sems):
        c = lax.axis_index("c")
        s = lax.axis_index("s")

        if edge_split:
            my_src = src_hbm.at[c * NS + s]
            my_dst = dst_hbm.at[c * NS + s]
        else:
            my_src = src_hbm.at[s]
            my_dst = dst_hbm.at[s]

        z16 = jnp.zeros((L,), jnp.float32)

        for p in range(npass):
            blk = c if npass == 1 else c * npass + p
            tab = tab_hbm if edge_split else tab_hbm.at[blk]

            def zbody(i, carry):
                for u in range(cb // L):
                    rows_v[0, i, pl.ds(u * L, L)] = z16
                return carry

            lax.fori_loop(0, CHUNK, zbody, 0)
            for z in range(zrows):
                pltpu.sync_copy(
                    rows_v.at[0],
                    acc_sh.at[pl.ds((s * zrows + z) * CHUNK, CHUNK)])
            plsc.subcore_barrier()

            def grp(g, carry):
                pltpu.sync_copy(my_src.at[pl.ds(g * _GRP, _GRP)], src_v)
                pltpu.sync_copy(my_dst.at[pl.ds(g * _GRP, _GRP)], dst_v)
                pltpu.async_copy(tab.at[src_v.at[0]], rows_v.at[0], sems.at[0])

                def body(j, carry2):
                    for b in (0, 1):
                        @pl.when(j % 2 == b)
                        def _():
                            pltpu.make_async_copy(
                                tab.at[src_v.at[j]], rows_v.at[b],
                                sems.at[b]).wait()

                            @pl.when(j + 1 < _GRP)
                            def _():
                                pltpu.async_copy(
                                    tab.at[src_v.at[j + 1]], rows_v.at[1 - b],
                                    sems.at[1 - b])

                            pltpu.sync_copy(rows_v.at[b],
                                            acc_sh.at[dst_v.at[j]], add=True)
                    return carry2

                lax.fori_loop(0, _GRP, body, 0)
                return carry

            lax.fori_loop(0, ngrp, grp, 0)
            plsc.subcore_barrier()
            pltpu.sync_copy(
                acc_sh.at[pl.ds(s * ROWS_PER_TILE, ROWS_PER_TILE)],
                out_hbm.at[blk].at[pl.ds(s * ROWS_PER_TILE, ROWS_PER_TILE)])

    return agg_kernel(table2, src3, dst3)


_ROWB = 1024
_NBLK1 = NHID // CHUNK


def _tca_body(degp_ref, x_ref, w1_ref, dinv_ref, xw_ref, xws_ref):
    deg = degp_ref[:, 0:1] + degp_ref[:, 1:2] + 1.0
    dinv = lax.rsqrt(deg)
    dinv_ref[...] = dinv
    xw = jnp.dot(x_ref[...], w1_ref[...], preferred_element_type=jnp.float32)
    xw_ref[...] = xw
    xws = dinv * xw
    for p in range(_NBLK1):
        xws_ref[p] = xws[:, p * CHUNK:(p + 1) * CHUNK]


def _tc_prescale(degp_t, x_pad, W1):
    ng = NACC // _ROWB
    return pl.pallas_call(
        _tca_body,
        grid=(ng,),
        in_specs=[
            pl.BlockSpec((_ROWB, NC), lambda i: (i, 0)),
            pl.BlockSpec((_ROWB, NFEAT), lambda i: (i, 0)),
            pl.BlockSpec((NFEAT, NHID), lambda i: (0, 0)),
        ],
        out_specs=(
            pl.BlockSpec((_ROWB, 1), lambda i: (i, 0)),
            pl.BlockSpec((_ROWB, NHID), lambda i: (i, 0)),
            pl.BlockSpec((_NBLK1, _ROWB, CHUNK), lambda i: (0, i, 0)),
        ),
        out_shape=(
            jax.ShapeDtypeStruct((NACC, 1), jnp.float32),
            jax.ShapeDtypeStruct((NACC, NHID), jnp.float32),
            jax.ShapeDtypeStruct((_NBLK1, NACC, CHUNK), jnp.float32),
        ),
    )(degp_t, x_pad, W1)


def _tcb_body(a1_ref, xw_ref, dinv_ref, b1_ref, w2_ref, hs_ref, hw_ref):
    dinv = dinv_ref[...]
    a1 = jnp.concatenate([a1_ref[p] for p in range(_NBLK1)], axis=1)
    t = dinv * a1 + (dinv * dinv) * xw_ref[...] + b1_ref[...]
    h = jax.nn.relu(t)
    hw = jnp.dot(h, w2_ref[...], preferred_element_type=jnp.float32)
    hw_ref[...] = hw
    hs_ref[...] = dinv * hw


def _tc_dense(a1, xw, dinv, b1, W2):
    ng = NACC // _ROWB
    return pl.pallas_call(
        _tcb_body,
        grid=(ng,),
        in_specs=[
            pl.BlockSpec((_NBLK1, _ROWB, CHUNK), lambda i: (0, i, 0)),
            pl.BlockSpec((_ROWB, NHID), lambda i: (i, 0)),
            pl.BlockSpec((_ROWB, 1), lambda i: (i, 0)),
            pl.BlockSpec((1, NHID), lambda i: (0, 0)),
            pl.BlockSpec((NHID, NOUT), lambda i: (0, 0)),
        ],
        out_specs=(
            pl.BlockSpec((_ROWB, NOUT), lambda i: (i, 0)),
            pl.BlockSpec((_ROWB, NOUT), lambda i: (i, 0)),
        ),
        out_shape=(
            jax.ShapeDtypeStruct((NACC, NOUT), jnp.float32),
            jax.ShapeDtypeStruct((NACC, NOUT), jnp.float32),
        ),
    )(a1, xw, dinv, b1, W2)


def _tcc0_body(a2_ref, hw_ref, dinv_ref, b2_ref, emb_ref):
    dinv = dinv_ref[...]
    a2 = a2_ref[0] + a2_ref[1]
    emb_ref[...] = dinv * a2 + (dinv * dinv) * hw_ref[...] + b2_ref[...]


def _tc_embeds(a2p, hw, dinv, b2):
    blk = 2000
    return pl.pallas_call(
        _tcc0_body,
        grid=(N // blk,),
        in_specs=[
            pl.BlockSpec((NC, blk, NOUT), lambda i: (0, i, 0)),
            pl.BlockSpec((blk, NOUT), lambda i: (i, 0)),
            pl.BlockSpec((blk, 1), lambda i: (i, 0)),
            pl.BlockSpec((1, NOUT), lambda i: (0, 0)),
        ],
        out_specs=pl.BlockSpec((blk, NOUT), lambda i: (i, 0)),
        out_shape=jax.ShapeDtypeStruct((N, NOUT), jnp.float32),
    )(a2p, hw, dinv, b2)


def _tcc_body(nit_ref, emb_in_ref, mu_ref, r_ref, dist_ref):
    embeds = emb_in_ref[...]

    norm = jnp.sqrt(jnp.sum(embeds * embeds, axis=1, keepdims=True))
    data = embeds / norm

    rows_iota = lax.broadcasted_iota(jnp.int32, (N, 1), 0)
    k_iota = lax.broadcasted_iota(jnp.int32, (K, 1), 0)
    c0 = data[0:1, :]
    diff = data - c0
    min_d0 = jnp.sum(diff * diff, axis=1, keepdims=True)
    mu0 = jnp.where(k_iota == 0, c0, jnp.zeros((K, NOUT), jnp.float32))

    def init_body(i, carry):
        mu, min_d = carry
        m = jnp.max(min_d)
        idx = jnp.min(jnp.where(min_d == m, rows_iota, N))
        sel = (rows_iota == idx).astype(jnp.float32)
        ci = jnp.sum(sel * data, axis=0, keepdims=True)
        mu = jnp.where(k_iota == i, ci, mu)
        d = data - ci
        min_d = jnp.minimum(min_d, jnp.sum(d * d, axis=1, keepdims=True))
        return mu, min_d

    mu, _ = lax.fori_loop(1, K, init_body, (mu0, min_d0))

    def body(_, mu):
        dist = lax.dot_general(data, mu, (((1,), (1,)), ((), ())),
                               preferred_element_type=jnp.float32)
        zmax = jnp.max(CLUSTER_TEMP * dist, axis=1, keepdims=True)
        e = jnp.exp(CLUSTER_TEMP * dist - zmax)
        r = e / jnp.sum(e, axis=1, keepdims=True)
        cluster_r = jnp.sum(r, axis=0, keepdims=True)
        cluster_mean = lax.dot_general(r, data, (((0,), (0,)), ((), ())),
                                       preferred_element_type=jnp.float32)
        return cluster_mean / cluster_r.T

    mu = lax.fori_loop(0, nit_ref[0], body, mu)
    dist = lax.dot_general(data, mu, (((1,), (1,)), ((), ())),
                           preferred_element_type=jnp.float32)
    zmax = jnp.max(CLUSTER_TEMP * dist, axis=1, keepdims=True)
    e = jnp.exp(CLUSTER_TEMP * dist - zmax)
    r = e / jnp.sum(e, axis=1, keepdims=True)
    mu_ref[...] = mu
    r_ref[...] = r
    dist_ref[...] = dist


def _tc_cluster(num_iter, embeds):
    return pl.pallas_call(
        _tcc_body,
        in_specs=[
            pl.BlockSpec(memory_space=pltpu.SMEM),
            pl.BlockSpec(),
        ],
        out_shape=(
            jax.ShapeDtypeStruct((K, NOUT), jnp.float32),
            jax.ShapeDtypeStruct((N, K), jnp.float32),
            jax.ShapeDtypeStruct((N, K), jnp.float32),
        ),
    )(num_iter, embeds)


def kernel(x, edge_index, num_iter, W1, b1, W2, b2):
    src = edge_index[0].astype(jnp.int32)
    dst = edge_index[1].astype(jnp.int32)
    e = src.shape[0]
    ep = NC * NS * ((e + NC * NS * CHUNK - 1) // (NC * NS * CHUNK)) * CHUNK
    pad = ep - e
    src_p = jnp.concatenate([src, jnp.zeros((pad,), jnp.int32)])
    dst_p = jnp.concatenate([dst, jnp.full((pad,), N, jnp.int32)])
    dst_deg = dst_p.reshape(NC * NS, -1, CHUNK)
    src_agg = src_p.reshape(NS, -1, CHUNK)
    dst_agg = dst_p.reshape(NS, -1, CHUNK)
    src_half = src_p.reshape(NC * NS, -1, CHUNK)
    dst_half = dst_p.reshape(NC * NS, -1, CHUNK)

    x_pad = jnp.concatenate(
        [x, jnp.zeros((NACC - N, x.shape[1]), x.dtype)], axis=0)

    degp = _sc_degree(dst_deg)
    dinv, xw, xws4 = _tc_prescale(degp.T, x_pad, W1)
    a1 = _sc_aggregate(xws4, src_agg, dst_agg,
                       nblk=_NBLK1)
    hs, hw = _tc_dense(a1, xw, dinv,
                       b1.reshape(1, NHID), W2)
    a2p = _sc_aggregate(hs, src_half, dst_half,
                        edge_split=True)

    embeds = _tc_embeds(a2p[:, :N, :], hw[:N], dinv[:N], b2.reshape(1, NOUT))
    nit = jnp.asarray(num_iter, jnp.int32).reshape(1)
    mu, r, dist = _tc_cluster(nit, embeds)
    return mu, r, embeds, dist
